# baseline (speedup 1.0000x reference)
import functools

import jax
import jax.numpy as jnp
from jax.experimental import pallas as pl
from jax.experimental.pallas import tpu as pltpu

_NEG = float(jnp.finfo(jnp.float32).min)
_INDEX_TOPK = 2048


def _sort_group(keys, idxs, lane, W):
    F = len(keys)
    log2w = W.bit_length() - 1
    for p in range(1, log2w + 1):
        k2 = 1 << p
        for q2 in range(p - 1, -1, -1):
            j = 1 << q2
            if j >= 128:
                jf, k2f = j // 128, k2 // 128
                nk, ni = list(keys), list(idxs)
                for f in range(F):
                    fp = f ^ jf
                    sk, si = keys[f], idxs[f]
                    pk, pi = keys[fp], idxs[fp]
                    pw = (pk > sk) | ((pk == sk) & (pi < si))
                    inv = (((f & jf) != 0) ^ ((f & k2f) != 0)
                           if k2f <= F else ((f & jf) != 0))
                    a, b = (sk, pk) if inv else (pk, sk)
                    ai, bi = (si, pi) if inv else (pi, si)
                    nk[f] = jnp.where(pw, a, b)
                    ni[f] = jnp.where(pw, ai, bi)
                keys, idxs = nk, ni
            else:
                if k2 <= 64:
                    m = ((lane & j) != 0) ^ ((lane & k2) != 0)
                else:
                    m = (lane & j) != 0
                lower = (lane & j) == 0
                nk, ni = [], []
                for f in range(F):
                    sk, si = keys[f], idxs[f]
                    pk = jnp.where(lower, jnp.roll(sk, -j, axis=1),
                                   jnp.roll(sk, j, axis=1))
                    pi = jnp.where(lower, jnp.roll(si, -j, axis=1),
                                   jnp.roll(si, j, axis=1))
                    pw = (pk > sk) | ((pk == sk) & (pi < si))
                    take = pw ^ m
                    if k2 >= 128 and (f & (k2 // 128)) != 0:
                        nk.append(jnp.where(take, sk, pk))
                        ni.append(jnp.where(take, si, pi))
                    else:
                        nk.append(jnp.where(take, pk, sk))
                        ni.append(jnp.where(take, pi, si))
                keys, idxs = nk, ni
    return keys, idxs


def _body(scal_ref, q_ref, k_ref, w_ref, idx_out_ref, val_out_ref,
          *, R, W, T, H, D, S, ratio, row0, G):
    b = pl.program_id(0)
    seqlen = scal_ref[0]
    offset = scal_ref[1]
    F = W // 128

    q = q_ref[0].reshape(R * H, D)
    km = k_ref[0]
    s = jax.lax.dot_general(q, km, (((1,), (1,)), ((), ())),
                            preferred_element_type=jnp.float32)
    s = jnp.maximum(s, 0.0).reshape(R, H, W) * w_ref[0][:, :, None]
    s = s.sum(axis=1)

    row = row0 + b * R + jax.lax.broadcasted_iota(jnp.int32, (R, W), 0)
    colR = jax.lax.broadcasted_iota(jnp.int32, (R, W), 1)
    thresh = (row + (seqlen - S) + 1) // ratio
    s = jnp.where(colR >= thresh, _NEG, s)

    lane = jax.lax.broadcasted_iota(jnp.int32, (G, 128), 1)
    for g in range(R // G):
        r0 = g * G
        keys = [jax.lax.slice(s, (r0, f * 128), (r0 + G, (f + 1) * 128))
                for f in range(F)]
        idxs = [f * 128 + lane for f in range(F)]
        keys, idxs = _sort_group(keys, idxs, lane, W)
        tG = jax.lax.slice(thresh, (r0, 0), (r0 + G, 1))
        for f in range(F):
            c0 = f * 128
            val_out_ref[0, r0:r0 + G, c0:c0 + 128] = keys[f]
            idx_out_ref[0, r0:r0 + G, c0:c0 + 128] = jnp.where(
                idxs[f] >= tG, -1, idxs[f] + offset)
    if W < T:
        val_out_ref[0, :, W:] = jnp.full((R, T - W), _NEG, jnp.float32)
        idx_out_ref[0, :, W:] = jnp.full((R, T - W), -1, jnp.int32)


def _band_call(q, k, w, scal, row0, rows, R, W, T, H, D, S, ratio, G,
               interpret=False):
    NB = rows // R
    B = q.shape[0]
    grid_spec = pltpu.PrefetchScalarGridSpec(
        num_scalar_prefetch=1,
        grid=(NB,),
        in_specs=[
            pl.BlockSpec((1, R, H, D), lambda b, s_ref: (0, b, 0, 0)),
            pl.BlockSpec((1, W, D), lambda b, s_ref: (0, 0, 0)),
            pl.BlockSpec((1, R, H), lambda b, s_ref: (0, b, 0)),
        ],
        out_specs=[
            pl.BlockSpec((1, R, T), lambda b, s_ref: (0, b, 0)),
            pl.BlockSpec((1, R, T), lambda b, s_ref: (0, b, 0)),
        ],
    )
    body = functools.partial(_body, R=R, W=W, T=T, H=H, D=D, S=S,
                             ratio=ratio, row0=row0, G=G)
    qs = jax.lax.slice_in_dim(q, row0, row0 + rows, axis=1)
    ws = jax.lax.slice_in_dim(w, row0, row0 + rows, axis=1)
    ks = jax.lax.slice_in_dim(k, 0, W, axis=1)
    return pl.pallas_call(
        body,
        grid_spec=grid_spec,
        out_shape=[
            jax.ShapeDtypeStruct((B, rows, T), jnp.int32),
            jax.ShapeDtypeStruct((B, rows, T), jnp.float32),
        ],
        interpret=interpret,
    )(scal, qs, ks, ws)


def _run(q_indexer, k_indexer, weights, seqlen, offset, interpret=False):
    B, S, H, D = q_indexer.shape
    T = k_indexer.shape[1]
    ratio = S // T
    k_out = min(_INDEX_TOPK, S // ratio)
    assert k_out == T, "kernel assumes full-width top_k (k == t)"

    scal = jnp.stack([jnp.asarray(seqlen, jnp.int32),
                      jnp.asarray(offset, jnp.int32)])

    bands = []
    row0, W = 0, 256
    while row0 < S:
        rows = (S if W >= T else min(S, W * ratio)) - row0
        Wc = min(W, T)
        R = min(128, rows)
        while rows % R:
            R //= 2
        bands.append((row0, rows, R, Wc))
        row0 += rows
        W *= 2

    idx_parts, val_parts = [], []
    for (row0, rows, R, W) in bands:
        i_p, v_p = _band_call(q_indexer, k_indexer, weights, scal,
                              row0, rows, R, W, T, H, D, S, ratio, 16,
                              interpret=interpret)
        idx_parts.append(i_p)
        val_parts.append(v_p)
    idx = jnp.concatenate(idx_parts, axis=1)
    val = jnp.concatenate(val_parts, axis=1)
    return idx, val


def kernel(q_indexer, k_indexer, weights, seqlen, offset):
    return _run(q_indexer, k_indexer, weights, seqlen, offset)


# speedup vs baseline: 1.2472x; 1.2472x over previous
import functools

import jax
import jax.numpy as jnp
from jax.experimental import pallas as pl
from jax.experimental.pallas import tpu as pltpu

_NEG = float(jnp.finfo(jnp.float32).min)
_INDEX_TOPK = 2048


def _sort_group(keys, idxs, lane, W):
    F = len(keys)
    log2w = W.bit_length() - 1
    for p in range(1, log2w + 1):
        k2 = 1 << p
        for q2 in range(p - 1, -1, -1):
            j = 1 << q2
            if j >= 128:
                jf, k2f = j // 128, k2 // 128
                nk, ni = list(keys), list(idxs)
                for f in range(F):
                    fp = f ^ jf
                    sk, si = keys[f], idxs[f]
                    pk, pi = keys[fp], idxs[fp]
                    pw = (pk > sk) | ((pk == sk) & (pi < si))
                    inv = (((f & jf) != 0) ^ ((f & k2f) != 0)
                           if k2f <= F else ((f & jf) != 0))
                    a, b = (sk, pk) if inv else (pk, sk)
                    ai, bi = (si, pi) if inv else (pi, si)
                    nk[f] = jnp.where(pw, a, b)
                    ni[f] = jnp.where(pw, ai, bi)
                keys, idxs = nk, ni
            else:
                if k2 <= 64:
                    m = ((lane & j) != 0) ^ ((lane & k2) != 0)
                else:
                    m = (lane & j) != 0
                lower = (lane & j) == 0
                nk, ni = [], []
                for f in range(F):
                    sk, si = keys[f], idxs[f]
                    pk = jnp.where(lower, jnp.roll(sk, -j, axis=1),
                                   jnp.roll(sk, j, axis=1))
                    pi = jnp.where(lower, jnp.roll(si, -j, axis=1),
                                   jnp.roll(si, j, axis=1))
                    pw = (pk > sk) | ((pk == sk) & (pi < si))
                    take = pw ^ m
                    if k2 >= 128 and (f & (k2 // 128)) != 0:
                        nk.append(jnp.where(take, sk, pk))
                        ni.append(jnp.where(take, si, pi))
                    else:
                        nk.append(jnp.where(take, pk, sk))
                        ni.append(jnp.where(take, pi, si))
                keys, idxs = nk, ni
    return keys, idxs


def _body(scal_ref, q_ref, k_ref, w_ref, idx_out_ref, val_out_ref,
          *, R, W, T, H, D, S, ratio, row0, G):
    b = pl.program_id(0)
    seqlen = scal_ref[0]
    offset = scal_ref[1]
    F = W // 128

    q = q_ref[0].reshape(R * H, D)
    km = k_ref[0]
    s = jax.lax.dot_general(q, km, (((1,), (1,)), ((), ())),
                            preferred_element_type=jnp.float32)
    s = jnp.maximum(s, 0.0).reshape(R, H, W) * w_ref[0][:, :, None]
    s = s.sum(axis=1)

    row = row0 + b * R + jax.lax.broadcasted_iota(jnp.int32, (R, W), 0)
    colR = jax.lax.broadcasted_iota(jnp.int32, (R, W), 1)
    thresh = (row + (seqlen - S) + 1) // ratio
    s = jnp.where(colR >= thresh, _NEG, s)

    lane = jax.lax.broadcasted_iota(jnp.int32, (G, 128), 1)
    for g in range(R // G):
        r0 = g * G
        keys = [jax.lax.slice(s, (r0, f * 128), (r0 + G, (f + 1) * 128))
                for f in range(F)]
        idxs = [f * 128 + lane for f in range(F)]
        keys, idxs = _sort_group(keys, idxs, lane, W)
        tG = jax.lax.slice(thresh, (r0, 0), (r0 + G, 1))
        for f in range(F):
            c0 = f * 128
            val_out_ref[0, r0:r0 + G, c0:c0 + 128] = keys[f]
            idx_out_ref[0, r0:r0 + G, c0:c0 + 128] = jnp.where(
                idxs[f] >= tG, -1, idxs[f] + offset)
    if W < T:
        val_out_ref[0, :, W:] = jnp.full((R, T - W), _NEG, jnp.float32)
        idx_out_ref[0, :, W:] = jnp.full((R, T - W), -1, jnp.int32)


def _band_call(q, k, w, scal, row0, rows, R, W, T, H, D, S, ratio, G,
               interpret=False):
    NB = rows // R
    B = q.shape[0]
    grid_spec = pltpu.PrefetchScalarGridSpec(
        num_scalar_prefetch=1,
        grid=(NB,),
        in_specs=[
            pl.BlockSpec((1, R, H, D), lambda b, s_ref: (0, b, 0, 0)),
            pl.BlockSpec((1, W, D), lambda b, s_ref: (0, 0, 0)),
            pl.BlockSpec((1, R, H), lambda b, s_ref: (0, b, 0)),
        ],
        out_specs=[
            pl.BlockSpec((1, R, T), lambda b, s_ref: (0, b, 0)),
            pl.BlockSpec((1, R, T), lambda b, s_ref: (0, b, 0)),
        ],
    )
    body = functools.partial(_body, R=R, W=W, T=T, H=H, D=D, S=S,
                             ratio=ratio, row0=row0, G=G)
    qs = jax.lax.slice_in_dim(q, row0, row0 + rows, axis=1)
    ws = jax.lax.slice_in_dim(w, row0, row0 + rows, axis=1)
    ks = jax.lax.slice_in_dim(k, 0, W, axis=1)
    return pl.pallas_call(
        body,
        grid_spec=grid_spec,
        out_shape=[
            jax.ShapeDtypeStruct((B, rows, T), jnp.int32),
            jax.ShapeDtypeStruct((B, rows, T), jnp.float32),
        ],
        interpret=interpret,
    )(scal, qs, ks, ws)


def _run(q_indexer, k_indexer, weights, seqlen, offset, interpret=False):
    B, S, H, D = q_indexer.shape
    T = k_indexer.shape[1]
    ratio = S // T
    k_out = min(_INDEX_TOPK, S // ratio)
    assert k_out == T, "kernel assumes full-width top_k (k == t)"

    scal = jnp.stack([jnp.asarray(seqlen, jnp.int32),
                      jnp.asarray(offset, jnp.int32)])

    bands = []
    row0, W = 0, 256
    while row0 < S:
        rows = (S if W >= T else min(S, W * ratio)) - row0
        Wc = min(W, T)
        R = min(64, rows)
        while rows % R:
            R //= 2
        bands.append((row0, rows, R, Wc))
        row0 += rows
        W *= 2

    idx_parts, val_parts = [], []
    for (row0, rows, R, W) in bands:
        i_p, v_p = _band_call(q_indexer, k_indexer, weights, scal,
                              row0, rows, R, W, T, H, D, S, ratio, 32,
                              interpret=interpret)
        idx_parts.append(i_p)
        val_parts.append(v_p)
    idx = jnp.concatenate(idx_parts, axis=1)
    val = jnp.concatenate(val_parts, axis=1)
    return idx, val


def kernel(q_indexer, k_indexer, weights, seqlen, offset):
    return _run(q_indexer, k_indexer, weights, seqlen, offset)


# speedup vs baseline: 1.2629x; 1.0126x over previous
import functools

import jax
import jax.numpy as jnp
from jax.experimental import pallas as pl
from jax.experimental.pallas import tpu as pltpu

_NEG = float(jnp.finfo(jnp.float32).min)
_INDEX_TOPK = 2048


def _sort_group(keys, idxs, lane, W):
    F = len(keys)
    log2w = W.bit_length() - 1
    for p in range(1, log2w + 1):
        k2 = 1 << p
        for q2 in range(p - 1, -1, -1):
            j = 1 << q2
            if j >= 128:
                jf, k2f = j // 128, k2 // 128
                nk, ni = list(keys), list(idxs)
                for f in range(F):
                    fp = f ^ jf
                    sk, si = keys[f], idxs[f]
                    pk, pi = keys[fp], idxs[fp]
                    pw = (pk > sk) | ((pk == sk) & (pi < si))
                    inv = (((f & jf) != 0) ^ ((f & k2f) != 0)
                           if k2f <= F else ((f & jf) != 0))
                    a, b = (sk, pk) if inv else (pk, sk)
                    ai, bi = (si, pi) if inv else (pi, si)
                    nk[f] = jnp.where(pw, a, b)
                    ni[f] = jnp.where(pw, ai, bi)
                keys, idxs = nk, ni
            else:
                if k2 <= 64:
                    m = ((lane & j) != 0) ^ ((lane & k2) != 0)
                else:
                    m = (lane & j) != 0
                lower = (lane & j) == 0
                nk, ni = [], []
                for f in range(F):
                    sk, si = keys[f], idxs[f]
                    pk = jnp.where(lower, jnp.roll(sk, -j, axis=1),
                                   jnp.roll(sk, j, axis=1))
                    pi = jnp.where(lower, jnp.roll(si, -j, axis=1),
                                   jnp.roll(si, j, axis=1))
                    pw = (pk > sk) | ((pk == sk) & (pi < si))
                    take = pw ^ m
                    if k2 >= 128 and (f & (k2 // 128)) != 0:
                        nk.append(jnp.where(take, sk, pk))
                        ni.append(jnp.where(take, si, pi))
                    else:
                        nk.append(jnp.where(take, pk, sk))
                        ni.append(jnp.where(take, pi, si))
                keys, idxs = nk, ni
    return keys, idxs


def _body(scal_ref, q_ref, k_ref, w_ref, idx_out_ref, val_out_ref,
          *, R, W, T, H, D, S, ratio, row0, G):
    b = pl.program_id(0)
    seqlen = scal_ref[0]
    offset = scal_ref[1]
    F = W // 128

    q = q_ref[0].reshape(R * H, D)
    km = k_ref[0]
    s = jax.lax.dot_general(q, km, (((1,), (1,)), ((), ())),
                            preferred_element_type=jnp.float32)
    s = jnp.maximum(s, 0.0).reshape(R, H, W) * w_ref[0][:, :, None]
    s = s.sum(axis=1)

    row = row0 + b * R + jax.lax.broadcasted_iota(jnp.int32, (R, W), 0)
    colR = jax.lax.broadcasted_iota(jnp.int32, (R, W), 1)
    thresh = (row + (seqlen - S) + 1) // ratio
    s = jnp.where(colR >= thresh, _NEG, s)

    lane = jax.lax.broadcasted_iota(jnp.int32, (G, 128), 1)
    for g in range(R // G):
        r0 = g * G
        keys = [jax.lax.slice(s, (r0, f * 128), (r0 + G, (f + 1) * 128))
                for f in range(F)]
        idxs = [f * 128 + lane for f in range(F)]
        keys, idxs = _sort_group(keys, idxs, lane, W)
        tG = jax.lax.slice(thresh, (r0, 0), (r0 + G, 1))
        for f in range(F):
            c0 = f * 128
            val_out_ref[0, r0:r0 + G, c0:c0 + 128] = keys[f]
            idx_out_ref[0, r0:r0 + G, c0:c0 + 128] = jnp.where(
                idxs[f] >= tG, -1, idxs[f] + offset)
    if W < T:
        val_out_ref[0, :, W:] = jnp.full((R, T - W), _NEG, jnp.float32)
        idx_out_ref[0, :, W:] = jnp.full((R, T - W), -1, jnp.int32)


def _band_call(q, k, w, scal, row0, rows, R, W, T, H, D, S, ratio, G,
               interpret=False):
    NB = rows // R
    B = q.shape[0]
    grid_spec = pltpu.PrefetchScalarGridSpec(
        num_scalar_prefetch=1,
        grid=(NB,),
        in_specs=[
            pl.BlockSpec((1, R, H, D), lambda b, s_ref: (0, b, 0, 0)),
            pl.BlockSpec((1, W, D), lambda b, s_ref: (0, 0, 0)),
            pl.BlockSpec((1, R, H), lambda b, s_ref: (0, b, 0)),
        ],
        out_specs=[
            pl.BlockSpec((1, R, T), lambda b, s_ref: (0, b, 0)),
            pl.BlockSpec((1, R, T), lambda b, s_ref: (0, b, 0)),
        ],
    )
    body = functools.partial(_body, R=R, W=W, T=T, H=H, D=D, S=S,
                             ratio=ratio, row0=row0, G=G)
    qs = jax.lax.slice_in_dim(q, row0, row0 + rows, axis=1)
    ws = jax.lax.slice_in_dim(w, row0, row0 + rows, axis=1)
    ks = jax.lax.slice_in_dim(k, 0, W, axis=1)
    return pl.pallas_call(
        body,
        grid_spec=grid_spec,
        out_shape=[
            jax.ShapeDtypeStruct((B, rows, T), jnp.int32),
            jax.ShapeDtypeStruct((B, rows, T), jnp.float32),
        ],
        interpret=interpret,
    )(scal, qs, ks, ws)


def _run(q_indexer, k_indexer, weights, seqlen, offset, interpret=False):
    B, S, H, D = q_indexer.shape
    T = k_indexer.shape[1]
    ratio = S // T
    k_out = min(_INDEX_TOPK, S // ratio)
    assert k_out == T, "kernel assumes full-width top_k (k == t)"

    scal = jnp.stack([jnp.asarray(seqlen, jnp.int32),
                      jnp.asarray(offset, jnp.int32)])

    bands = []
    row0, W = 0, 256
    while row0 < S:
        rows = (S if W >= T else min(S, W * ratio)) - row0
        Wc = min(W, T)
        R = min(64, rows)
        while rows % R:
            R //= 2
        bands.append((row0, rows, R, Wc))
        row0 += rows
        W *= 2

    idx_parts, val_parts = [], []
    for (row0, rows, R, W) in bands:
        i_p, v_p = _band_call(q_indexer, k_indexer, weights, scal,
                              row0, rows, R, W, T, H, D, S, ratio, 64,
                              interpret=interpret)
        idx_parts.append(i_p)
        val_parts.append(v_p)
    idx = jnp.concatenate(idx_parts, axis=1)
    val = jnp.concatenate(val_parts, axis=1)
    return idx, val


def kernel(q_indexer, k_indexer, weights, seqlen, offset):
    return _run(q_indexer, k_indexer, weights, seqlen, offset)


# speedup vs baseline: 1.3663x; 1.0819x over previous
import functools

import jax
import jax.numpy as jnp
from jax.experimental import pallas as pl
from jax.experimental.pallas import tpu as pltpu

_NEG = float(jnp.finfo(jnp.float32).min)
_INDEX_TOPK = 2048


def _sort_group(keys, idxs, lane, W):
    F = len(keys)
    log2w = W.bit_length() - 1
    for p in range(1, log2w + 1):
        k2 = 1 << p
        for q2 in range(p - 1, -1, -1):
            j = 1 << q2
            if j >= 128:
                jf, k2f = j // 128, k2 // 128
                nk, ni = list(keys), list(idxs)
                for f in range(F):
                    fp = f ^ jf
                    sk, si = keys[f], idxs[f]
                    pk, pi = keys[fp], idxs[fp]
                    pw = (pk > sk) | ((pk == sk) & (pi < si))
                    inv = (((f & jf) != 0) ^ ((f & k2f) != 0)
                           if k2f <= F else ((f & jf) != 0))
                    a, b = (sk, pk) if inv else (pk, sk)
                    ai, bi = (si, pi) if inv else (pi, si)
                    nk[f] = jnp.where(pw, a, b)
                    ni[f] = jnp.where(pw, ai, bi)
                keys, idxs = nk, ni
            else:
                if k2 <= 64:
                    m = ((lane & j) != 0) ^ ((lane & k2) != 0)
                else:
                    m = (lane & j) != 0
                lower = (lane & j) == 0
                nk, ni = [], []
                for f in range(F):
                    sk, si = keys[f], idxs[f]
                    pk = jnp.where(lower, jnp.roll(sk, -j, axis=1),
                                   jnp.roll(sk, j, axis=1))
                    pi = jnp.where(lower, jnp.roll(si, -j, axis=1),
                                   jnp.roll(si, j, axis=1))
                    pw = (pk > sk) | ((pk == sk) & (pi < si))
                    take = pw ^ m
                    if k2 >= 128 and (f & (k2 // 128)) != 0:
                        nk.append(jnp.where(take, sk, pk))
                        ni.append(jnp.where(take, si, pi))
                    else:
                        nk.append(jnp.where(take, pk, sk))
                        ni.append(jnp.where(take, pi, si))
                keys, idxs = nk, ni
    return keys, idxs


def _body(scal_ref, q_ref, k_ref, w_ref, *rest,
          R, W, T, H, D, S, ratio, row0, G, has_prev):
    if has_prev:
        _pi, _pv, idx_out_ref, val_out_ref = rest
    else:
        idx_out_ref, val_out_ref = rest
    b = pl.program_id(0)
    seqlen = scal_ref[0]
    offset = scal_ref[1]
    F = W // 128

    q = q_ref[0].reshape(R * H, D)
    km = k_ref[0]
    s = jax.lax.dot_general(q, km, (((1,), (1,)), ((), ())),
                            preferred_element_type=jnp.float32)
    s = jnp.maximum(s, 0.0).reshape(R, H, W) * w_ref[0][:, :, None]
    s = s.sum(axis=1)

    row = row0 + b * R + jax.lax.broadcasted_iota(jnp.int32, (R, W), 0)
    colR = jax.lax.broadcasted_iota(jnp.int32, (R, W), 1)
    thresh = (row + (seqlen - S) + 1) // ratio
    s = jnp.where(colR >= thresh, _NEG, s)

    lane = jax.lax.broadcasted_iota(jnp.int32, (G, 128), 1)
    for g in range(R // G):
        r0 = g * G
        keys = [jax.lax.slice(s, (r0, f * 128), (r0 + G, (f + 1) * 128))
                for f in range(F)]
        idxs = [f * 128 + lane for f in range(F)]
        keys, idxs = _sort_group(keys, idxs, lane, W)
        tG = jax.lax.slice(thresh, (r0, 0), (r0 + G, 1))
        for f in range(F):
            c0 = f * 128
            val_out_ref[0, r0:r0 + G, c0:c0 + 128] = keys[f]
            idx_out_ref[0, r0:r0 + G, c0:c0 + 128] = jnp.where(
                idxs[f] >= tG, -1, idxs[f] + offset)
    if W < T:
        val_out_ref[0, :, W:] = jnp.full((R, T - W), _NEG, jnp.float32)
        idx_out_ref[0, :, W:] = jnp.full((R, T - W), -1, jnp.int32)


def _band_call(q, k, w, scal, prev, row0, rows, R, W, T, H, D, S, ratio, G,
               interpret=False):
    NB = rows // R
    B = q.shape[0]
    b0 = row0 // R
    in_specs = [
        pl.BlockSpec((1, R, H, D), lambda b, s_ref: (0, b0 + b, 0, 0)),
        pl.BlockSpec((1, W, D), lambda b, s_ref: (0, 0, 0)),
        pl.BlockSpec((1, R, H), lambda b, s_ref: (0, b0 + b, 0)),
    ]
    n_in = 4
    if prev is not None:
        in_specs += [pl.BlockSpec(memory_space=pl.ANY),
                     pl.BlockSpec(memory_space=pl.ANY)]
        n_in = 6
    grid_spec = pltpu.PrefetchScalarGridSpec(
        num_scalar_prefetch=1,
        grid=(NB,),
        in_specs=in_specs,
        out_specs=[
            pl.BlockSpec((1, R, T), lambda b, s_ref: (0, b0 + b, 0)),
            pl.BlockSpec((1, R, T), lambda b, s_ref: (0, b0 + b, 0)),
        ],
    )
    body = functools.partial(_body, R=R, W=W, T=T, H=H, D=D, S=S,
                             ratio=ratio, row0=row0, G=G,
                             has_prev=prev is not None)
    args = (scal, q, k, w) + (tuple(prev) if prev is not None else ())
    return pl.pallas_call(
        body,
        grid_spec=grid_spec,
        out_shape=[
            jax.ShapeDtypeStruct((B, S, T), jnp.int32),
            jax.ShapeDtypeStruct((B, S, T), jnp.float32),
        ],
        input_output_aliases=({4: 0, 5: 1} if prev is not None else {}),
        interpret=interpret,
    )(*args)


def _run(q_indexer, k_indexer, weights, seqlen, offset, interpret=False):
    B, S, H, D = q_indexer.shape
    T = k_indexer.shape[1]
    ratio = S // T
    k_out = min(_INDEX_TOPK, S // ratio)
    assert k_out == T, "kernel assumes full-width top_k (k == t)"

    scal = jnp.stack([jnp.asarray(seqlen, jnp.int32),
                      jnp.asarray(offset, jnp.int32)])

    bands = []
    row0, W = 0, 256
    while row0 < S:
        rows = (S if W >= T else min(S, W * ratio)) - row0
        Wc = min(W, T)
        R = min(64, rows)
        while rows % R:
            R //= 2
        bands.append((row0, rows, R, Wc))
        row0 += rows
        W *= 2

    prev = None
    for (row0, rows, R, W) in bands:
        prev = _band_call(q_indexer, k_indexer, weights, scal, prev,
                          row0, rows, R, W, T, H, D, S, ratio, 64,
                          interpret=interpret)
    return prev[0], prev[1]


def kernel(q_indexer, k_indexer, weights, seqlen, offset):
    return _run(q_indexer, k_indexer, weights, seqlen, offset)


# speedup vs baseline: 1.3800x; 1.0100x over previous
import functools

import jax
import jax.numpy as jnp
from jax.experimental import pallas as pl
from jax.experimental.pallas import tpu as pltpu

_NEG = float(jnp.finfo(jnp.float32).min)
_INDEX_TOPK = 2048


def _sort_group(keys, idxs, lane, W):
    F = len(keys)
    log2w = W.bit_length() - 1
    for p in range(1, log2w + 1):
        k2 = 1 << p
        for q2 in range(p - 1, -1, -1):
            j = 1 << q2
            if j >= 128:
                jf, k2f = j // 128, k2 // 128
                nk, ni = list(keys), list(idxs)
                for f in range(F):
                    fp = f ^ jf
                    sk, si = keys[f], idxs[f]
                    pk, pi = keys[fp], idxs[fp]
                    pw = (pk > sk) | ((pk == sk) & (pi < si))
                    inv = (((f & jf) != 0) ^ ((f & k2f) != 0)
                           if k2f <= F else ((f & jf) != 0))
                    a, b = (sk, pk) if inv else (pk, sk)
                    ai, bi = (si, pi) if inv else (pi, si)
                    nk[f] = jnp.where(pw, a, b)
                    ni[f] = jnp.where(pw, ai, bi)
                keys, idxs = nk, ni
            else:
                if k2 <= 64:
                    m = ((lane & j) != 0) ^ ((lane & k2) != 0)
                else:
                    m = (lane & j) != 0
                lower = (lane & j) == 0
                nk, ni = [], []
                for f in range(F):
                    sk, si = keys[f], idxs[f]
                    pk = jnp.where(lower, jnp.roll(sk, -j, axis=1),
                                   jnp.roll(sk, j, axis=1))
                    pi = jnp.where(lower, jnp.roll(si, -j, axis=1),
                                   jnp.roll(si, j, axis=1))
                    pw = (pk > sk) | ((pk == sk) & (pi < si))
                    take = pw ^ m
                    if k2 >= 128 and (f & (k2 // 128)) != 0:
                        nk.append(jnp.where(take, sk, pk))
                        ni.append(jnp.where(take, si, pi))
                    else:
                        nk.append(jnp.where(take, pk, sk))
                        ni.append(jnp.where(take, pi, si))
                keys, idxs = nk, ni
    return keys, idxs


def _body(scal_ref, q_ref, k_ref, w_ref, *rest,
          R, W, T, H, D, S, ratio, row0, G, has_prev):
    if has_prev:
        _pi, _pv, idx_out_ref, val_out_ref = rest
    else:
        idx_out_ref, val_out_ref = rest
    b = pl.program_id(0)
    seqlen = scal_ref[0]
    offset = scal_ref[1]
    F = W // 128

    q = q_ref[0].reshape(R * H, D)
    km = k_ref[0]
    s = jax.lax.dot_general(q, km, (((1,), (1,)), ((), ())),
                            preferred_element_type=jnp.float32)
    s = jnp.maximum(s, 0.0).reshape(R, H, W) * w_ref[0][:, :, None]
    s = s.sum(axis=1)

    row = row0 + b * R + jax.lax.broadcasted_iota(jnp.int32, (R, W), 0)
    colR = jax.lax.broadcasted_iota(jnp.int32, (R, W), 1)
    thresh = (row + (seqlen - S) + 1) // ratio
    s = jnp.where(colR >= thresh, _NEG, s)

    lane = jax.lax.broadcasted_iota(jnp.int32, (G, 128), 1)
    for g in range(R // G):
        r0 = g * G
        keys = [jax.lax.slice(s, (r0, f * 128), (r0 + G, (f + 1) * 128))
                for f in range(F)]
        idxs = [f * 128 + lane for f in range(F)]
        keys, idxs = _sort_group(keys, idxs, lane, W)
        tG = jax.lax.slice(thresh, (r0, 0), (r0 + G, 1))
        for f in range(F):
            c0 = f * 128
            val_out_ref[0, r0:r0 + G, c0:c0 + 128] = keys[f]
            idx_out_ref[0, r0:r0 + G, c0:c0 + 128] = jnp.where(
                idxs[f] >= tG, -1, idxs[f] + offset)
    if W < T:
        val_out_ref[0, :, W:] = jnp.full((R, T - W), _NEG, jnp.float32)
        idx_out_ref[0, :, W:] = jnp.full((R, T - W), -1, jnp.int32)


def _band_call(q, k, w, scal, prev, row0, rows, R, W, T, H, D, S, ratio, G,
               interpret=False):
    NB = rows // R
    B = q.shape[0]
    b0 = row0 // R
    in_specs = [
        pl.BlockSpec((1, R, H, D), lambda b, s_ref: (0, b0 + b, 0, 0)),
        pl.BlockSpec((1, W, D), lambda b, s_ref: (0, 0, 0)),
        pl.BlockSpec((1, R, H), lambda b, s_ref: (0, b0 + b, 0)),
    ]
    n_in = 4
    if prev is not None:
        in_specs += [pl.BlockSpec(memory_space=pl.ANY),
                     pl.BlockSpec(memory_space=pl.ANY)]
        n_in = 6
    grid_spec = pltpu.PrefetchScalarGridSpec(
        num_scalar_prefetch=1,
        grid=(NB,),
        in_specs=in_specs,
        out_specs=[
            pl.BlockSpec((1, R, T), lambda b, s_ref: (0, b0 + b, 0)),
            pl.BlockSpec((1, R, T), lambda b, s_ref: (0, b0 + b, 0)),
        ],
    )
    body = functools.partial(_body, R=R, W=W, T=T, H=H, D=D, S=S,
                             ratio=ratio, row0=row0, G=G,
                             has_prev=prev is not None)
    args = (scal, q, k, w) + (tuple(prev) if prev is not None else ())
    return pl.pallas_call(
        body,
        grid_spec=grid_spec,
        out_shape=[
            jax.ShapeDtypeStruct((B, S, T), jnp.int32),
            jax.ShapeDtypeStruct((B, S, T), jnp.float32),
        ],
        input_output_aliases=({4: 0, 5: 1} if prev is not None else {}),
        interpret=interpret,
    )(*args)


def _run(q_indexer, k_indexer, weights, seqlen, offset, interpret=False):
    B, S, H, D = q_indexer.shape
    T = k_indexer.shape[1]
    ratio = S // T
    k_out = min(_INDEX_TOPK, S // ratio)
    assert k_out == T, "kernel assumes full-width top_k (k == t)"

    scal = jnp.stack([jnp.asarray(seqlen, jnp.int32),
                      jnp.asarray(offset, jnp.int32)])

    bands = []
    row0, W = 0, 256
    while row0 < S:
        rows = (S if W >= T else min(S, W * ratio)) - row0
        Wc = min(W, T)
        R = min(64 if Wc >= 1024 else 128, rows)
        while rows % R:
            R //= 2
        bands.append((row0, rows, R, Wc))
        row0 += rows
        W *= 2

    prev = None
    for (row0, rows, R, W) in bands:
        prev = _band_call(q_indexer, k_indexer, weights, scal, prev,
                          row0, rows, R, W, T, H, D, S, ratio, 64,
                          interpret=interpret)
    return prev[0], prev[1]


def kernel(q_indexer, k_indexer, weights, seqlen, offset):
    return _run(q_indexer, k_indexer, weights, seqlen, offset)


# speedup vs baseline: 1.4538x; 1.0535x over previous
import functools

import jax
import jax.numpy as jnp
from jax.experimental import pallas as pl
from jax.experimental.pallas import tpu as pltpu

_NEG = float(jnp.finfo(jnp.float32).min)
_INDEX_TOPK = 2048


def _stage(keys, idxs, lane, j, k2, flip=False):
    F = len(keys)
    if j >= 128:
        jf, k2f = j // 128, k2 // 128
        nk, ni = list(keys), list(idxs)
        for f in range(F):
            fp = f ^ jf
            sk, si = keys[f], idxs[f]
            pk, pi = keys[fp], idxs[fp]
            pw = (pk > sk) | ((pk == sk) & (pi < si))
            inv = ((f & jf) != 0) ^ ((f & k2f) != 0) ^ flip
            a, b = (sk, pk) if inv else (pk, sk)
            ai, bi = (si, pi) if inv else (pi, si)
            nk[f] = jnp.where(pw, a, b)
            ni[f] = jnp.where(pw, ai, bi)
        return nk, ni
    if k2 <= 64:
        m = ((lane & j) != 0) ^ ((lane & k2) != 0)
    else:
        m = (lane & j) != 0
    lower = (lane & j) == 0
    nk, ni = [], []
    for f in range(F):
        sk, si = keys[f], idxs[f]
        pk = jnp.where(lower, jnp.roll(sk, -j, axis=1),
                       jnp.roll(sk, j, axis=1))
        pi = jnp.where(lower, jnp.roll(si, -j, axis=1),
                       jnp.roll(si, j, axis=1))
        pw = (pk > sk) | ((pk == sk) & (pi < si))
        take = pw ^ m
        if ((f & (k2 // 128)) != 0 if k2 >= 128 else False) ^ flip:
            nk.append(jnp.where(take, sk, pk))
            ni.append(jnp.where(take, si, pi))
        else:
            nk.append(jnp.where(take, pk, sk))
            ni.append(jnp.where(take, pi, si))
    return nk, ni


def _sort_group(keys, idxs, lane, W, flip=False):
    log2w = W.bit_length() - 1
    for p in range(1, log2w + 1):
        k2 = 1 << p
        for q2 in range(p - 1, -1, -1):
            keys, idxs = _stage(keys, idxs, lane, 1 << q2, k2, flip)
    return keys, idxs


def _sort_group_split(keys, idxs, lane, W):
    F = len(keys)
    Fh, Fq = F // 2, F // 4
    kh, ih = _sort_group(keys[:Fh], idxs[:Fh], lane, W // 2)
    kq, iq = _sort_group(keys[Fh:Fh + Fq], idxs[Fh:Fh + Fq], lane, W // 4,
                         flip=True)
    neg = jnp.full_like(keys[0], _NEG)
    keys = kh + [neg] * (Fh - Fq) + kq
    idxs = ih + idxs[Fh + Fq:] + iq
    k2 = W
    j = W // 2
    while j:
        keys, idxs = _stage(keys, idxs, lane, j, k2)
        j //= 2
    return keys, idxs


def _body(scal_ref, q_ref, k_ref, w_ref, *rest,
          R, W, T, H, D, S, ratio, row0, G, has_prev, split):
    if has_prev:
        _pi, _pv, idx_out_ref, val_out_ref = rest
    else:
        idx_out_ref, val_out_ref = rest
    b = pl.program_id(0)
    seqlen = scal_ref[0]
    offset = scal_ref[1]
    F = W // 128

    q = q_ref[0].reshape(R * H, D)
    km = k_ref[0]
    s = jax.lax.dot_general(q, km, (((1,), (1,)), ((), ())),
                            preferred_element_type=jnp.float32)
    s = jnp.maximum(s, 0.0).reshape(R, H, W) * w_ref[0][:, :, None]
    s = s.sum(axis=1)

    row = row0 + b * R + jax.lax.broadcasted_iota(jnp.int32, (R, W), 0)
    colR = jax.lax.broadcasted_iota(jnp.int32, (R, W), 1)
    thresh = (row + (seqlen - S) + 1) // ratio
    s = jnp.where(colR >= thresh, _NEG, s)

    lane = jax.lax.broadcasted_iota(jnp.int32, (G, 128), 1)
    for g in range(R // G):
        r0 = g * G
        keys = [jax.lax.slice(s, (r0, f * 128), (r0 + G, (f + 1) * 128))
                for f in range(F)]
        idxs = [f * 128 + lane for f in range(F)]
        if split:
            keys, idxs = _sort_group_split(keys, idxs, lane, W)
        else:
            keys, idxs = _sort_group(keys, idxs, lane, W)
        tG = jax.lax.slice(thresh, (r0, 0), (r0 + G, 1))
        for f in range(F):
            c0 = f * 128
            val_out_ref[0, r0:r0 + G, c0:c0 + 128] = keys[f]
            idx_out_ref[0, r0:r0 + G, c0:c0 + 128] = jnp.where(
                idxs[f] >= tG, -1, idxs[f] + offset)
    if W < T:
        val_out_ref[0, :, W:] = jnp.full((R, T - W), _NEG, jnp.float32)
        idx_out_ref[0, :, W:] = jnp.full((R, T - W), -1, jnp.int32)


def _band_call(q, k, w, scal, prev, row0, rows, R, W, T, H, D, S, ratio, G,
               split, interpret=False):
    NB = rows // R
    B = q.shape[0]
    b0 = row0 // R
    in_specs = [
        pl.BlockSpec((1, R, H, D), lambda b, s_ref: (0, b0 + b, 0, 0)),
        pl.BlockSpec((1, W, D), lambda b, s_ref: (0, 0, 0)),
        pl.BlockSpec((1, R, H), lambda b, s_ref: (0, b0 + b, 0)),
    ]
    n_in = 4
    if prev is not None:
        in_specs += [pl.BlockSpec(memory_space=pl.ANY),
                     pl.BlockSpec(memory_space=pl.ANY)]
        n_in = 6
    grid_spec = pltpu.PrefetchScalarGridSpec(
        num_scalar_prefetch=1,
        grid=(NB,),
        in_specs=in_specs,
        out_specs=[
            pl.BlockSpec((1, R, T), lambda b, s_ref: (0, b0 + b, 0)),
            pl.BlockSpec((1, R, T), lambda b, s_ref: (0, b0 + b, 0)),
        ],
    )
    body = functools.partial(_body, R=R, W=W, T=T, H=H, D=D, S=S,
                             ratio=ratio, row0=row0, G=G,
                             has_prev=prev is not None, split=split)
    args = (scal, q, k, w) + (tuple(prev) if prev is not None else ())
    return pl.pallas_call(
        body,
        grid_spec=grid_spec,
        out_shape=[
            jax.ShapeDtypeStruct((B, S, T), jnp.int32),
            jax.ShapeDtypeStruct((B, S, T), jnp.float32),
        ],
        input_output_aliases=({4: 0, 5: 1} if prev is not None else {}),
        interpret=interpret,
    )(*args)


def _run(q_indexer, k_indexer, weights, seqlen, offset, interpret=False):
    B, S, H, D = q_indexer.shape
    T = k_indexer.shape[1]
    ratio = S // T
    k_out = min(_INDEX_TOPK, S // ratio)
    assert k_out == T, "kernel assumes full-width top_k (k == t)"

    scal = jnp.stack([jnp.asarray(seqlen, jnp.int32),
                      jnp.asarray(offset, jnp.int32)])

    bands = []
    row0, W = 0, 256
    while row0 < S:
        if W >= T:
            split_end = min(S, (3 * T * ratio) // 4)
            if T >= 512 and row0 < split_end:
                bands.append((row0, split_end - row0, 64, T, True))
                row0 = split_end
            if row0 < S:
                bands.append((row0, S - row0, 64, T, False))
            row0 = S
        else:
            rows = min(S, W * ratio) - row0
            R = min(128, rows)
            while rows % R:
                R //= 2
            bands.append((row0, rows, R, W, False))
            row0 += rows
        W *= 2

    prev = None
    for (row0, rows, R, W, split) in bands:
        prev = _band_call(q_indexer, k_indexer, weights, scal, prev,
                          row0, rows, R, W, T, H, D, S, ratio, 64, split,
                          interpret=interpret)
    return prev[0], prev[1]


def kernel(q_indexer, k_indexer, weights, seqlen, offset):
    return _run(q_indexer, k_indexer, weights, seqlen, offset)
